# baseline (speedup 1.0000x reference)
import jax
import jax.numpy as jnp
from jax.experimental import pallas as pl

N = 100000
D = 128
ROWS_PER_BLK = 2000
NBLK = N // ROWS_PER_BLK


def _score_body(x_ref, w_ref, b_ref, z_ref):
    xb = x_ref[...]
    w = w_ref[...]
    z = jnp.dot(xb, w) + b_ref[0, 0]
    z_ref[0, 0, :] = z[:, 0]


def _scores(x, W, b):
    wt = W.reshape(D, 1)
    b2 = b.reshape(1, 1)
    z3 = pl.pallas_call(
        _score_body,
        grid=(NBLK,),
        in_specs=[
            pl.BlockSpec((ROWS_PER_BLK, D), lambda i: (i, 0)),
            pl.BlockSpec((D, 1), lambda i: (0, 0)),
            pl.BlockSpec((1, 1), lambda i: (0, 0)),
        ],
        out_specs=pl.BlockSpec((1, 1, ROWS_PER_BLK), lambda i: (i, 0, 0)),
        out_shape=jax.ShapeDtypeStruct((NBLK, 1, ROWS_PER_BLK), jnp.float32),
    )(x, wt, b2)
    return z3.reshape(N)


def kernel(x, W, b):
    z = _scores(x, W, b)
    k = N // 4
    score = jax.nn.sigmoid(z)
    sel_s, idx = jax.lax.top_k(score, k)
    new_val = x[idx, :] * sel_s[:, None]
    return new_val, idx
